# baseline (speedup 1.0000x reference)
import jax
import jax.numpy as jnp
from jax import lax
from jax.experimental import pallas as pl
from jax.experimental.pallas import tpu as pltpu
from jax.experimental.pallas import tpu_sc as plsc

N_LEVELS = 16
N_MIN = 16.0
N_MAX = 512
HASH_EXP = 19
T = 2 ** HASH_EXP
MASK = T - 1
NUM_COORDS = 262144
PI2 = -1640531535

NC = 2
NS = 16
NW = NC * NS
NPW = NUM_COORDS // NW
C = 512
G = C // 16
NCHUNK = NPW // C


def _sc_body(x_hbm, y_hbm, sc_hbm, tbl_hbm, out_hbm,
             xv, yv, scv, idxv, valsv, rwv, cwv, outv, sem):
    wid = lax.axis_index("s") * jnp.int32(NC) + lax.axis_index("c")
    base = wid * jnp.int32(NPW)
    pltpu.sync_copy(sc_hbm, scv)
    lanes = lax.iota(jnp.int32, 16)

    def do_chunk(c, _):
        start = base + c * jnp.int32(C)
        pltpu.sync_copy(x_hbm.at[pl.ds(start, C)], xv)
        pltpu.sync_copy(y_hbm.at[pl.ds(start, C)], yv)

        for l in range(N_LEVELS):
            sv = scv[pl.ds(l * 16, 16)]
            lvl2 = jnp.int32(2 * l * T)

            def pass1(g, _):
                gb = g * jnp.int32(16)
                xf = xv[pl.ds(gb, 16)]
                yf = yv[pl.ds(gb, 16)]
                sx = xf * sv
                sy = yf * sv
                nx0 = sx.astype(jnp.int32)
                nx1 = (sx + jnp.float32(1.0)).astype(jnp.int32)
                ny0 = sy.astype(jnp.int32)
                ny1 = (sy + jnp.float32(1.0)).astype(jnp.int32)
                rwv[pl.ds(gb, 16)] = sx - nx0.astype(jnp.float32)
                cwv[pl.ds(gb, 16)] = sy - ny0.astype(jnp.float32)
                hy0 = ny0 * jnp.int32(PI2)
                hy1 = ny1 * jnp.int32(PI2)
                mk = jnp.int32(MASK)
                one = jnp.int32(1)
                i00 = (nx0 ^ hy0) & mk
                i01 = (nx0 ^ hy1) & mk
                i11 = (nx1 ^ hy1) & mk
                i10 = (nx1 ^ hy0) & mk
                for k, ii in enumerate((i00, i01, i11, i10)):
                    e0 = lax.shift_left(ii, one) + lvl2
                    idxv[pl.ds(gb + jnp.int32(2 * k * C), 16)] = e0
                    idxv[pl.ds(gb + jnp.int32((2 * k + 1) * C), 16)] = e0 + one
                return _

            lax.fori_loop(jnp.int32(0), jnp.int32(G), pass1, None)

            pltpu.async_copy(tbl_hbm.at[idxv], valsv, sem).wait()

            col0 = jnp.full((16,), 2 * l, jnp.int32)
            col1 = jnp.full((16,), 2 * l + 1, jnp.int32)

            def pass2(g, _):
                gb = g * jnp.int32(16)
                rw = rwv[pl.ds(gb, 16)]
                cw = cwv[pl.ds(gb, 16)]
                ridx = lanes + gb
                a = jnp.float32(1.0) - cw
                b_ = jnp.float32(1.0) - rw
                obase = ridx * jnp.int32(32)
                for f, colc in ((0, col0), (1, col1)):
                    v00 = valsv[pl.ds(gb + jnp.int32((0 + f) * C), 16)]
                    v01 = valsv[pl.ds(gb + jnp.int32((2 + f) * C), 16)]
                    v11 = valsv[pl.ds(gb + jnp.int32((4 + f) * C), 16)]
                    v10 = valsv[pl.ds(gb + jnp.int32((6 + f) * C), 16)]
                    o = (((v00 * a) * b_ + (v01 * cw) * b_)
                         + (v10 * a) * rw) + (v11 * cw) * rw
                    plsc.store_scatter(outv, [obase + colc], o)
                return _

            lax.fori_loop(jnp.int32(0), jnp.int32(G), pass2, None)

        pltpu.sync_copy(outv, out_hbm.at[pl.ds(start * jnp.int32(32), C * 32)])
        return _

    lax.fori_loop(jnp.int32(0), jnp.int32(NCHUNK), do_chunk, None)


@jax.jit
def _run(xf, yf, scales_b, tbl):
    mesh = plsc.VectorSubcoreMesh(core_axis_name="c", subcore_axis_name="s")
    k = pl.kernel(
        _sc_body,
        out_type=jax.ShapeDtypeStruct((NUM_COORDS * 2 * N_LEVELS,), jnp.float32),
        mesh=mesh,
        compiler_params=pltpu.CompilerParams(
            needs_layout_passes=False, use_tc_tiling_on_sc=False),
        scratch_types=[
            pltpu.VMEM((C,), jnp.float32),
            pltpu.VMEM((C,), jnp.float32),
            pltpu.VMEM((N_LEVELS * 16,), jnp.float32),
            pltpu.VMEM((8 * C,), jnp.int32),
            pltpu.VMEM((8 * C,), jnp.float32),
            pltpu.VMEM((C,), jnp.float32),
            pltpu.VMEM((C,), jnp.float32),
            pltpu.VMEM((C * 2 * N_LEVELS,), jnp.float32),
            pltpu.SemaphoreType.DMA,
        ],
    )
    return k(xf, yf, scales_b, tbl)


def kernel(input_coords, tables):
    coords_f = input_coords.astype(jnp.float32)
    xf = coords_f[:, 0]
    yf = coords_f[:, 1]
    b = jnp.exp((jnp.log(jnp.float32(N_MAX)) - jnp.log(jnp.float32(N_MIN)))
                / (N_LEVELS - 1))
    scales = jnp.stack(
        [jnp.floor(jnp.float32(N_MIN) * b ** i) / jnp.float32(N_MAX)
         for i in range(N_LEVELS)])
    scales_b = jnp.broadcast_to(scales[:, None], (N_LEVELS, 16)).reshape(-1)
    tbl = tables.reshape(N_LEVELS * T * 2)
    out = _run(xf, yf, scales_b, tbl)
    return out.reshape(NUM_COORDS, 2 * N_LEVELS)


# speedup vs baseline: 6.1162x; 6.1162x over previous
import jax
import jax.numpy as jnp
from jax import lax
from jax.experimental import pallas as pl
from jax.experimental.pallas import tpu as pltpu
from jax.experimental.pallas import tpu_sc as plsc

N_LEVELS = 16
N_MIN = 16.0
N_MAX = 512
HASH_EXP = 19
T = 2 ** HASH_EXP
MASK = T - 1
NUM_COORDS = 262144
PI2 = -1640531535

NC = 2
NS = 16
NW = NC * NS
NPW = NUM_COORDS // NW
C = 512
G = C // 16
NCHUNK = NPW // C


def _sc_body(x_hbm, y_hbm, sc_hbm, tbl_hbm, out_hbm,
             xv, yv, scv, idxv, valsv, rwv, cwv, outv, sem):
    wid = lax.axis_index("s") * jnp.int32(NC) + lax.axis_index("c")
    base = wid * jnp.int32(NPW)
    pltpu.sync_copy(sc_hbm, scv)
    lanes = lax.iota(jnp.int32, 16)

    def do_chunk(c, _):
        start = base + c * jnp.int32(C)
        pltpu.sync_copy(x_hbm.at[pl.ds(start, C)], xv)
        pltpu.sync_copy(y_hbm.at[pl.ds(start, C)], yv)

        for l in range(N_LEVELS):
            sv = scv[pl.ds(l * 16, 16)]
            lvlw = jnp.int32(2 * l * T)

            def pass1(g, _):
                gb = g * jnp.int32(16)
                xf = xv[pl.ds(gb, 16)]
                yf = yv[pl.ds(gb, 16)]
                sx = xf * sv
                sy = yf * sv
                nx0 = sx.astype(jnp.int32)
                nx1 = (sx + jnp.float32(1.0)).astype(jnp.int32)
                ny0 = sy.astype(jnp.int32)
                ny1 = (sy + jnp.float32(1.0)).astype(jnp.int32)
                rwv[pl.ds(gb, 16)] = sx - nx0.astype(jnp.float32)
                cwv[pl.ds(gb, 16)] = sy - ny0.astype(jnp.float32)
                hy0 = ny0 * jnp.int32(PI2)
                hy1 = ny1 * jnp.int32(PI2)
                mk = jnp.int32(MASK)
                c7 = jnp.int32(7)
                c8 = jnp.int32(8)
                c127 = jnp.int32(127)
                c128 = jnp.int32(128)
                i00 = (nx0 ^ hy0) & mk
                i01 = (nx0 ^ hy1) & mk
                i11 = (nx1 ^ hy1) & mk
                i10 = (nx1 ^ hy0) & mk
                for k, ii in enumerate((i00, i01, i11, i10)):
                    e0 = (lax.shift_left(lax.shift_right_logical(ii, c7), c8)
                          | (ii & c127)) + lvlw
                    idxv[pl.ds(gb + jnp.int32(2 * k * C), 16)] = e0
                    idxv[pl.ds(gb + jnp.int32((2 * k + 1) * C), 16)] = e0 + c128
                return _

            lax.fori_loop(jnp.int32(0), jnp.int32(G), pass1, None)

            pltpu.async_copy(tbl_hbm.at[idxv], valsv, sem).wait()

            col0 = jnp.full((16,), 2 * l, jnp.int32)
            col1 = jnp.full((16,), 2 * l + 1, jnp.int32)

            def pass2(g, _):
                gb = g * jnp.int32(16)
                rw = rwv[pl.ds(gb, 16)]
                cw = cwv[pl.ds(gb, 16)]
                ridx = lanes + gb
                a = jnp.float32(1.0) - cw
                b_ = jnp.float32(1.0) - rw
                obase = ridx * jnp.int32(32)
                for f, colc in ((0, col0), (1, col1)):
                    v00 = valsv[pl.ds(gb + jnp.int32((0 + f) * C), 16)]
                    v01 = valsv[pl.ds(gb + jnp.int32((2 + f) * C), 16)]
                    v11 = valsv[pl.ds(gb + jnp.int32((4 + f) * C), 16)]
                    v10 = valsv[pl.ds(gb + jnp.int32((6 + f) * C), 16)]
                    o = (((v00 * a) * b_ + (v01 * cw) * b_)
                         + (v10 * a) * rw) + (v11 * cw) * rw
                    plsc.store_scatter(outv, [obase + colc], o)
                return _

            lax.fori_loop(jnp.int32(0), jnp.int32(G), pass2, None)

        pltpu.sync_copy(outv, out_hbm.at[pl.ds(start * jnp.int32(32), C * 32)])
        return _

    lax.fori_loop(jnp.int32(0), jnp.int32(NCHUNK), do_chunk, None)


@jax.jit
def _run(xf, yf, scales_b, tbl):
    mesh = plsc.VectorSubcoreMesh(core_axis_name="c", subcore_axis_name="s")
    k = pl.kernel(
        _sc_body,
        out_type=jax.ShapeDtypeStruct((NUM_COORDS * 2 * N_LEVELS,), jnp.float32),
        mesh=mesh,
        compiler_params=pltpu.CompilerParams(
            needs_layout_passes=False, use_tc_tiling_on_sc=False),
        scratch_types=[
            pltpu.VMEM((C,), jnp.float32),
            pltpu.VMEM((C,), jnp.float32),
            pltpu.VMEM((N_LEVELS * 16,), jnp.float32),
            pltpu.VMEM((8 * C,), jnp.int32),
            pltpu.VMEM((8 * C,), jnp.float32),
            pltpu.VMEM((C,), jnp.float32),
            pltpu.VMEM((C,), jnp.float32),
            pltpu.VMEM((C * 2 * N_LEVELS,), jnp.float32),
            pltpu.SemaphoreType.DMA,
        ],
    )
    return k(xf, yf, scales_b, tbl)


def kernel(input_coords, tables):
    coords_f = input_coords.astype(jnp.float32)
    xf = coords_f[:, 0]
    yf = coords_f[:, 1]
    b = jnp.exp((jnp.log(jnp.float32(N_MAX)) - jnp.log(jnp.float32(N_MIN)))
                / (N_LEVELS - 1))
    scales = jnp.stack(
        [jnp.floor(jnp.float32(N_MIN) * b ** i) / jnp.float32(N_MAX)
         for i in range(N_LEVELS)])
    scales_b = jnp.broadcast_to(scales[:, None], (N_LEVELS, 16)).reshape(-1)
    tbl = tables.reshape(N_LEVELS, T // 128, 128, 2)
    tbl = tbl.transpose(0, 1, 3, 2).reshape(N_LEVELS * T * 2)
    out = _run(xf, yf, scales_b, tbl)
    return out.reshape(NUM_COORDS, 2 * N_LEVELS)


# speedup vs baseline: 6.3169x; 1.0328x over previous
import jax
import jax.numpy as jnp
from jax import lax
from jax.experimental import pallas as pl
from jax.experimental.pallas import tpu as pltpu
from jax.experimental.pallas import tpu_sc as plsc

N_LEVELS = 16
N_MIN = 16.0
N_MAX = 512
HASH_EXP = 19
T = 2 ** HASH_EXP
MASK = T - 1
NUM_COORDS = 262144
PI2 = -1640531535

NC = 2
NS = 16
NW = NC * NS
NPW = NUM_COORDS // NW
C = 512
G = C // 16
NCHUNK = NPW // C


def _sc_body(x_hbm, y_hbm, sc_hbm, tbl_hbm, out_hbm,
             xv, yv, scv, idxv0, idxv1, valsv0, valsv1,
             rwv0, rwv1, cwv0, cwv1, outv, sem0, sem1):
    wid = lax.axis_index("s") * jnp.int32(NC) + lax.axis_index("c")
    base = wid * jnp.int32(NPW)
    pltpu.sync_copy(sc_hbm, scv)
    lanes = lax.iota(jnp.int32, 16)
    lanes32 = lanes * jnp.int32(32)
    idxbufs = (idxv0, idxv1)
    valbufs = (valsv0, valsv1)
    rwbufs = (rwv0, rwv1)
    cwbufs = (cwv0, cwv1)
    sems = (sem0, sem1)

    def hash_pass(l, idxv, rwv, cwv):
        sv = scv[pl.ds(l * 16, 16)]
        lvlw = jnp.int32(2 * l * T)

        def pass1(g, carry):
            gb = g * jnp.int32(16)
            xf = xv[pl.ds(gb, 16)]
            yf = yv[pl.ds(gb, 16)]
            sx = xf * sv
            sy = yf * sv
            nx0 = sx.astype(jnp.int32)
            nx1 = (sx + jnp.float32(1.0)).astype(jnp.int32)
            ny0 = sy.astype(jnp.int32)
            ny1 = (sy + jnp.float32(1.0)).astype(jnp.int32)
            rwv[pl.ds(gb, 16)] = sx - nx0.astype(jnp.float32)
            cwv[pl.ds(gb, 16)] = sy - ny0.astype(jnp.float32)
            hy0 = ny0 * jnp.int32(PI2)
            hy1 = ny1 * jnp.int32(PI2)
            mk = jnp.int32(MASK)
            c7 = jnp.int32(7)
            c8 = jnp.int32(8)
            c127 = jnp.int32(127)
            c128 = jnp.int32(128)
            i00 = (nx0 ^ hy0) & mk
            i01 = (nx0 ^ hy1) & mk
            i11 = (nx1 ^ hy1) & mk
            i10 = (nx1 ^ hy0) & mk
            for k, ii in enumerate((i00, i01, i11, i10)):
                e0 = (lax.shift_left(lax.shift_right_logical(ii, c7), c8)
                      | (ii & c127)) + lvlw
                idxv[pl.ds(gb + jnp.int32(2 * k * C), 16)] = e0
                idxv[pl.ds(gb + jnp.int32((2 * k + 1) * C), 16)] = e0 + c128
            return carry

        lax.fori_loop(jnp.int32(0), jnp.int32(G), pass1, None)

    def interp_pass(l, valsv, rwv, cwv):
        col0 = jnp.full((16,), 2 * l, jnp.int32)
        col1 = jnp.full((16,), 2 * l + 1, jnp.int32)

        def pass2(g, carry):
            gb = g * jnp.int32(16)
            rw = rwv[pl.ds(gb, 16)]
            cw = cwv[pl.ds(gb, 16)]
            a = jnp.float32(1.0) - cw
            b_ = jnp.float32(1.0) - rw
            obase = lanes32 + gb * jnp.int32(32)
            for f, colc in ((0, col0), (1, col1)):
                v00 = valsv[pl.ds(gb + jnp.int32((0 + f) * C), 16)]
                v01 = valsv[pl.ds(gb + jnp.int32((2 + f) * C), 16)]
                v11 = valsv[pl.ds(gb + jnp.int32((4 + f) * C), 16)]
                v10 = valsv[pl.ds(gb + jnp.int32((6 + f) * C), 16)]
                o = (((v00 * a) * b_ + (v01 * cw) * b_)
                     + (v10 * a) * rw) + (v11 * cw) * rw
                plsc.store_scatter(outv, [obase + colc], o)
            return carry

        lax.fori_loop(jnp.int32(0), jnp.int32(G), pass2, None)

    def do_chunk(c, _):
        start = base + c * jnp.int32(C)
        pltpu.sync_copy(x_hbm.at[pl.ds(start, C)], xv)
        pltpu.sync_copy(y_hbm.at[pl.ds(start, C)], yv)

        copies = [None, None]
        for l in range(N_LEVELS):
            p = l & 1
            q = (l - 1) & 1
            hash_pass(l, idxbufs[p], rwbufs[p], cwbufs[p])
            if l > 0:
                copies[q].wait()
                interp_pass(l - 1, valbufs[q], rwbufs[q], cwbufs[q])
            copies[p] = pltpu.async_copy(
                tbl_hbm.at[idxbufs[p]], valbufs[p], sems[p])
        q = (N_LEVELS - 1) & 1
        copies[q].wait()
        interp_pass(N_LEVELS - 1, valbufs[q], rwbufs[q], cwbufs[q])

        pltpu.sync_copy(outv, out_hbm.at[pl.ds(start * jnp.int32(32), C * 32)])
        return _

    lax.fori_loop(jnp.int32(0), jnp.int32(NCHUNK), do_chunk, None)


@jax.jit
def _run(xf, yf, scales_b, tbl):
    mesh = plsc.VectorSubcoreMesh(core_axis_name="c", subcore_axis_name="s")
    k = pl.kernel(
        _sc_body,
        out_type=jax.ShapeDtypeStruct((NUM_COORDS * 2 * N_LEVELS,), jnp.float32),
        mesh=mesh,
        compiler_params=pltpu.CompilerParams(
            needs_layout_passes=False, use_tc_tiling_on_sc=False),
        scratch_types=[
            pltpu.VMEM((C,), jnp.float32),
            pltpu.VMEM((C,), jnp.float32),
            pltpu.VMEM((N_LEVELS * 16,), jnp.float32),
            pltpu.VMEM((8 * C,), jnp.int32),
            pltpu.VMEM((8 * C,), jnp.int32),
            pltpu.VMEM((8 * C,), jnp.float32),
            pltpu.VMEM((8 * C,), jnp.float32),
            pltpu.VMEM((C,), jnp.float32),
            pltpu.VMEM((C,), jnp.float32),
            pltpu.VMEM((C,), jnp.float32),
            pltpu.VMEM((C,), jnp.float32),
            pltpu.VMEM((C * 2 * N_LEVELS,), jnp.float32),
            pltpu.SemaphoreType.DMA,
            pltpu.SemaphoreType.DMA,
        ],
    )
    return k(xf, yf, scales_b, tbl)


def kernel(input_coords, tables):
    coords_f = input_coords.astype(jnp.float32)
    xf = coords_f[:, 0]
    yf = coords_f[:, 1]
    b = jnp.exp((jnp.log(jnp.float32(N_MAX)) - jnp.log(jnp.float32(N_MIN)))
                / (N_LEVELS - 1))
    scales = jnp.stack(
        [jnp.floor(jnp.float32(N_MIN) * b ** i) / jnp.float32(N_MAX)
         for i in range(N_LEVELS)])
    scales_b = jnp.broadcast_to(scales[:, None], (N_LEVELS, 16)).reshape(-1)
    tbl = tables.reshape(N_LEVELS, T // 128, 128, 2)
    tbl = tbl.transpose(0, 1, 3, 2).reshape(N_LEVELS * T * 2)
    out = _run(xf, yf, scales_b, tbl)
    return out.reshape(NUM_COORDS, 2 * N_LEVELS)
